# baseline (speedup 1.0000x reference)
import functools

import jax
import jax.numpy as jnp
from jax import lax
from jax.experimental import pallas as pl
from jax.experimental.pallas import tpu as pltpu
from jax.experimental.pallas import tpu_sc as plsc

_NC = 2
_NS = 16
_NW = _NC * _NS
_IDX_CHUNK = 128


@functools.lru_cache(maxsize=None)
def _make_gather_sum(B, CTX, V, D):
    b_per_w = B // _NW
    n_idx = b_per_w * CTX
    n_chunks = n_idx // _IDX_CHUNK
    mesh = plsc.VectorSubcoreMesh(core_axis_name="c", subcore_axis_name="s")

    @functools.partial(
        pl.kernel,
        mesh=mesh,
        out_type=jax.ShapeDtypeStruct((B, D), jnp.float32),
        compiler_params=pltpu.CompilerParams(use_tc_tiling_on_sc=False),
        scratch_types=[
            pltpu.VMEM((n_idx,), jnp.int32),
            pltpu.VMEM((n_idx, D), jnp.float32),
            pltpu.VMEM((b_per_w, D), jnp.float32),
            pltpu.SemaphoreType.DMA,
        ],
    )
    def gather_sum(idx_hbm, table_hbm, out_hbm, idx_v, rows_v, acc_v, sem):
        wid = lax.axis_index("s") * _NC + lax.axis_index("c")
        pltpu.sync_copy(idx_hbm.at[pl.ds(wid * n_idx, n_idx)], idx_v)
        copies = [
            pltpu.async_copy(
                table_hbm.at[idx_v.at[pl.ds(t * _IDX_CHUNK, _IDX_CHUNK)]],
                rows_v.at[pl.ds(t * _IDX_CHUNK, _IDX_CHUNK)],
                sem,
            )
            for t in range(n_chunks)
        ]
        for cp in copies:
            cp.wait()

        def row_body(j, carry):
            base_r = j * CTX
            for l in range(D // 16):
                sl = pl.ds(l * 16, 16)
                acc = rows_v[base_r, sl]
                for c in range(1, CTX):
                    acc = acc + rows_v[base_r + c, sl]
                acc_v[j, sl] = acc
            return carry

        lax.fori_loop(0, b_per_w, row_body, 0)
        pltpu.sync_copy(acc_v, out_hbm.at[pl.ds(wid * b_per_w, b_per_w)])

    return gather_sum


def _tc_body(nv, BV, V, x_ref, w_ref, b_ref, o_ref, m_scr, s_scr):
    p = pl.program_id(0)
    v = pl.program_id(1)
    logits = lax.dot_general(
        x_ref[...], w_ref[...],
        (((1,), (1,)), ((), ())),
        preferred_element_type=jnp.float32,
    ) + b_ref[...]

    @pl.when(jnp.logical_and(p == 0, v == 0))
    def _init():
        m_scr[...] = jnp.full(m_scr.shape, -1e30, jnp.float32)
        s_scr[...] = jnp.zeros(s_scr.shape, jnp.float32)

    @pl.when(p == 0)
    def _stats():
        col = v * BV + lax.broadcasted_iota(jnp.int32, logits.shape, 1)
        lm = jnp.where(col < V, logits, -1e30)
        m_old = m_scr[...]
        m_new = jnp.maximum(m_old, jnp.max(lm, axis=1, keepdims=True))
        s_scr[...] = (
            s_scr[...] * jnp.exp(m_old - m_new)
            + jnp.sum(jnp.exp(lm - m_new), axis=1, keepdims=True)
        )
        m_scr[...] = m_new

    @pl.when(p == 1)
    def _write():
        o_ref[...] = logits - (m_scr[...] + jnp.log(s_scr[...]))


def _fused_proj_logsoftmax(x, W, b, BV=1024):
    B, D = x.shape
    V = W.shape[0]
    nv = pl.cdiv(V, BV)
    return pl.pallas_call(
        functools.partial(_tc_body, nv, BV, V),
        grid=(2, nv),
        in_specs=[
            pl.BlockSpec((B, D), lambda p, v: (0, 0)),
            pl.BlockSpec((BV, D), lambda p, v: (v, 0)),
            pl.BlockSpec((1, BV), lambda p, v: (0, v)),
        ],
        out_specs=pl.BlockSpec((B, BV), lambda p, v: (0, jnp.where(p == 0, 0, v))),
        out_shape=jax.ShapeDtypeStruct((B, V), jnp.float32),
        scratch_shapes=[
            pltpu.VMEM((B, 1), jnp.float32),
            pltpu.VMEM((B, 1), jnp.float32),
        ],
    )(x, W, b.reshape(1, V))


def kernel(inputs, emb, W, b):
    B, CTX = inputs.shape
    V, D = emb.shape
    idx = inputs.reshape(-1).astype(jnp.int32)
    x = _make_gather_sum(B, CTX, V, D)(idx, emb)
    return _fused_proj_logsoftmax(x, W, b)


# speedup vs baseline: 1.0817x; 1.0817x over previous
import functools

import jax
import jax.numpy as jnp
from jax import lax
from jax.experimental import pallas as pl
from jax.experimental.pallas import tpu as pltpu
from jax.experimental.pallas import tpu_sc as plsc

_NC = 2
_NS = 16
_NW = _NC * _NS
_IDX_CHUNK = 128


@functools.lru_cache(maxsize=None)
def _make_gather_sum(B, CTX, V, D):
    b_per_w = B // _NW
    n_idx = b_per_w * CTX
    n_chunks = n_idx // _IDX_CHUNK
    mesh = plsc.VectorSubcoreMesh(core_axis_name="c", subcore_axis_name="s")

    @functools.partial(
        pl.kernel,
        mesh=mesh,
        out_type=jax.ShapeDtypeStruct((B, D), jnp.float32),
        compiler_params=pltpu.CompilerParams(use_tc_tiling_on_sc=False),
        scratch_types=[
            pltpu.VMEM((n_idx,), jnp.int32),
            pltpu.VMEM((n_idx, D), jnp.float32),
            pltpu.VMEM((b_per_w, D), jnp.float32),
            pltpu.SemaphoreType.DMA,
        ],
    )
    def gather_sum(idx_hbm, table_hbm, out_hbm, idx_v, rows_v, acc_v, sem):
        wid = lax.axis_index("s") * _NC + lax.axis_index("c")
        pltpu.sync_copy(idx_hbm.at[pl.ds(wid * n_idx, n_idx)], idx_v)
        copies = [
            pltpu.async_copy(
                table_hbm.at[idx_v.at[pl.ds(t * _IDX_CHUNK, _IDX_CHUNK)]],
                rows_v.at[pl.ds(t * _IDX_CHUNK, _IDX_CHUNK)],
                sem,
            )
            for t in range(n_chunks)
        ]
        for cp in copies:
            cp.wait()

        def row_body(j, carry):
            base_r = j * CTX
            for l in range(D // 16):
                sl = pl.ds(l * 16, 16)
                acc = rows_v[base_r, sl]
                for c in range(1, CTX):
                    acc = acc + rows_v[base_r + c, sl]
                acc_v[j, sl] = acc
            return carry

        lax.fori_loop(0, b_per_w, row_body, 0)
        pltpu.sync_copy(acc_v, out_hbm.at[pl.ds(wid * b_per_w, b_per_w)])

    return gather_sum


def _tc_body(nv, BV, V, x_ref, w_ref, b_ref, o_ref, m_scr, s_scr):
    p = pl.program_id(0)
    v = pl.program_id(1)
    logits = lax.dot_general(
        x_ref[...], w_ref[...],
        (((1,), (1,)), ((), ())),
        preferred_element_type=jnp.float32,
    ) + b_ref[...]

    @pl.when(jnp.logical_and(p == 0, v == 0))
    def _init():
        m_scr[...] = jnp.full(m_scr.shape, -1e30, jnp.float32)
        s_scr[...] = jnp.zeros(s_scr.shape, jnp.float32)

    @pl.when(p == 0)
    def _stats():
        col = v * BV + lax.broadcasted_iota(jnp.int32, logits.shape, 1)
        lm = jnp.where(col < V, logits, -1e30)
        m_old = m_scr[...]
        m_new = jnp.maximum(m_old, jnp.max(lm, axis=1, keepdims=True))
        s_scr[...] = (
            s_scr[...] * jnp.exp(m_old - m_new)
            + jnp.sum(jnp.exp(lm - m_new), axis=1, keepdims=True)
        )
        m_scr[...] = m_new

    @pl.when(p == 1)
    def _write():
        o_ref[...] = logits - (m_scr[...] + jnp.log(s_scr[...]))


def _fused_proj_logsoftmax(x, W, b, BV=2048):
    B, D = x.shape
    V = W.shape[0]
    nv = pl.cdiv(V, BV)
    return pl.pallas_call(
        functools.partial(_tc_body, nv, BV, V),
        grid=(2, nv),
        in_specs=[
            pl.BlockSpec((B, D), lambda p, v: (0, 0)),
            pl.BlockSpec((BV, D), lambda p, v: (v, 0)),
            pl.BlockSpec((1, BV), lambda p, v: (0, v)),
        ],
        out_specs=pl.BlockSpec((B, BV), lambda p, v: (0, jnp.where(p == 0, 0, v))),
        out_shape=jax.ShapeDtypeStruct((B, V), jnp.float32),
        scratch_shapes=[
            pltpu.VMEM((B, 1), jnp.float32),
            pltpu.VMEM((B, 1), jnp.float32),
        ],
    )(x, W, b.reshape(1, V))


def kernel(inputs, emb, W, b):
    B, CTX = inputs.shape
    V, D = emb.shape
    idx = inputs.reshape(-1).astype(jnp.int32)
    x = _make_gather_sum(B, CTX, V, D)(idx, emb)
    return _fused_proj_logsoftmax(
        x.astype(jnp.bfloat16), W.astype(jnp.bfloat16), b)
